# Initial kernel scaffold; baseline (speedup 1.0000x reference)
import functools

import jax
import jax.numpy as jnp
from jax import lax
from jax.experimental import pallas as pl
from jax.experimental.pallas import tpu as pltpu
from jax.experimental.pallas import tpu_sc as plsc

N = 10000
E = 320000
NFEAT = 128
NHID = 64
H = 2 * NHID
MAX_LOGSTD = 10.0

NC = 2
NS = 16
NW = NC * NS
CHUNK = 128
EPAD = NW * CHUNK * ((E + NW * CHUNK - 1) // (NW * CHUNK))
TPE = EPAD // NW
NCH = TPE // CHUNK
NACC = 10240
RPT = NACC // NS
BLK = 128
NBLK = NACC // BLK

_mesh = plsc.VectorSubcoreMesh(core_axis_name="c", subcore_axis_name="s")


@functools.partial(
    pl.kernel,
    out_type=jax.ShapeDtypeStruct((NC, NACC, 16), jnp.float32),
    mesh=_mesh,
    scratch_types=[
        pltpu.VMEM_SHARED((NACC, 16), jnp.float32),
        pltpu.VMEM((CHUNK,), jnp.int32),
        pltpu.VMEM((CHUNK, 16), jnp.float32),
    ],
)
def _deg_kernel(dst_hbm, ones_hbm, zeros_hbm, out_hbm, acc, idx_v, ones_v):
    c = lax.axis_index("c")
    s = lax.axis_index("s")
    wid = s * NC + c
    pltpu.sync_copy(zeros_hbm, acc.at[pl.ds(s * RPT, RPT)])
    pltpu.sync_copy(ones_hbm, ones_v)
    plsc.subcore_barrier()

    def body(j, carry):
        base = wid * TPE + j * CHUNK
        pltpu.sync_copy(dst_hbm.at[pl.ds(base, CHUNK)], idx_v)
        pltpu.sync_copy(ones_v, acc.at[idx_v], add=True)
        return carry

    lax.fori_loop(0, NCH, body, 0)
    plsc.subcore_barrier()
    pltpu.sync_copy(acc.at[pl.ds(s * RPT, RPT)],
                    out_hbm.at[c, pl.ds(s * RPT, RPT)])


@functools.partial(
    pl.kernel,
    out_type=jax.ShapeDtypeStruct((NC, NACC, H), jnp.float32),
    mesh=_mesh,
    scratch_types=[
        pltpu.VMEM_SHARED((NACC, H), jnp.float32),
        pltpu.VMEM((CHUNK,), jnp.int32),
        pltpu.VMEM((CHUNK,), jnp.int32),
        pltpu.VMEM((CHUNK, H), jnp.float32),
        pltpu.SemaphoreType.DMA,
    ],
)
def _adj_kernel(y_hbm, src_hbm, dst_hbm, zeros_hbm, out_hbm,
                acc, sidx, didx, rows, gsem):
    c = lax.axis_index("c")
    s = lax.axis_index("s")
    wid = s * NC + c
    pltpu.sync_copy(zeros_hbm, acc.at[pl.ds(s * RPT, RPT)])
    plsc.subcore_barrier()

    def body(j, carry):
        base = wid * TPE + j * CHUNK
        pltpu.sync_copy(src_hbm.at[pl.ds(base, CHUNK)], sidx)
        pltpu.sync_copy(dst_hbm.at[pl.ds(base, CHUNK)], didx)
        pltpu.async_copy(y_hbm.at[sidx], rows, gsem).wait()
        pltpu.sync_copy(rows, acc.at[didx], add=True)
        return carry

    lax.fori_loop(0, NCH, body, 0)
    plsc.subcore_barrier()
    pltpu.sync_copy(acc.at[pl.ds(s * RPT, RPT)],
                    out_hbm.at[c, pl.ds(s * RPT, RPT)])


def _tc1_body(x_ref, w_ref, b_ref, d_ref, y1_ref, dinv_ref):
    i = pl.program_id(0)
    deg = d_ref[0, :, :1] + d_ref[1, :, :1] + 1.0
    rows = i * BLK + lax.broadcasted_iota(jnp.int32, (BLK, 1), 0)
    dinv = jnp.where(rows < N, lax.rsqrt(deg), 0.0)
    h0 = jnp.dot(x_ref[...], w_ref[...], preferred_element_type=jnp.float32)
    y1_ref[...] = dinv * (h0 + b_ref[...])
    dinv_ref[...] = dinv


def _tc2_body(u_ref, y1_ref, dinv_ref, w_ref, b_ref, y2_ref):
    dinv = dinv_ref[...]
    s1 = dinv * (u_ref[0] + u_ref[1] + y1_ref[...])
    h = jnp.dot(s1, w_ref[...], preferred_element_type=jnp.float32)
    h = jnp.maximum(h + b_ref[...], 0.0)
    y2_ref[...] = dinv * h


def _tc3_body(u_ref, y2_ref, dinv_ref, w_ref, b_ref, eps_ref, z_ref):
    s2 = dinv_ref[...] * (u_ref[0] + u_ref[1] + y2_ref[...])
    o = jnp.dot(s2, w_ref[...], preferred_element_type=jnp.float32)
    o = o + b_ref[...]
    mu = o[:, :NHID]
    ls = jnp.minimum(o[:, NHID:], MAX_LOGSTD)
    z_ref[...] = mu + eps_ref[...] * jnp.exp(ls)


def _row_spec(width):
    return pl.BlockSpec((BLK, width), lambda i: (i, 0))


def _full_spec(shape):
    ndim = len(shape)
    return pl.BlockSpec(shape, lambda i: (0,) * ndim)


def _parts_spec(width):
    return pl.BlockSpec((NC, BLK, width), lambda i: (0, i, 0))


def kernel(x, edge_index, lin_W, lin_b, W1, b1, Wmu, bmu, Wls, bls, eps):
    src = edge_index[0]
    dst = edge_index[1]
    pad = jnp.full((EPAD - E,), N, dtype=jnp.int32)
    src_p = jnp.concatenate([src, pad])
    dst_p = jnp.concatenate([dst, pad])
    x_p = jnp.pad(x, ((0, NACC - N), (0, 0)))
    eps_p = jnp.pad(eps, ((0, NACC - N), (0, 0)))
    ones16 = jnp.ones((CHUNK, 16), jnp.float32)
    zeros16 = jnp.zeros((RPT, 16), jnp.float32)
    zrows = jnp.zeros((RPT, H), jnp.float32)
    W2 = jnp.concatenate([Wmu, Wls], axis=1)
    b2 = jnp.concatenate([bmu, bls])[None, :]
    b1r = b1[None, :]
    linbr = lin_b[None, :]

    deg_parts = _deg_kernel(dst_p, ones16, zeros16)

    y1, dinv = pl.pallas_call(
        _tc1_body,
        grid=(NBLK,),
        in_specs=[_row_spec(NFEAT), _full_spec((NFEAT, H)),
                  _full_spec((1, H)), _parts_spec(16)],
        out_specs=[_row_spec(H), _row_spec(1)],
        out_shape=[jax.ShapeDtypeStruct((NACC, H), jnp.float32),
                   jax.ShapeDtypeStruct((NACC, 1), jnp.float32)],
    )(x_p, lin_W, linbr, deg_parts)

    u1 = _adj_kernel(y1, src_p, dst_p, zrows)

    y2 = pl.pallas_call(
        _tc2_body,
        grid=(NBLK,),
        in_specs=[_parts_spec(H), _row_spec(H), _row_spec(1),
                  _full_spec((H, H)), _full_spec((1, H))],
        out_specs=_row_spec(H),
        out_shape=jax.ShapeDtypeStruct((NACC, H), jnp.float32),
    )(u1, y1, dinv, W1, b1r)

    u2 = _adj_kernel(y2, src_p, dst_p, zrows)

    z = pl.pallas_call(
        _tc3_body,
        grid=(NBLK,),
        in_specs=[_parts_spec(H), _row_spec(H), _row_spec(1),
                  _full_spec((H, H)), _full_spec((1, H)), _row_spec(NHID)],
        out_specs=_row_spec(NHID),
        out_shape=jax.ShapeDtypeStruct((NACC, NHID), jnp.float32),
    )(u2, y2, dinv, W2, b2, eps_p)

    return z[:N]


# speedup vs baseline: 11.2263x; 11.2263x over previous
import functools

import jax
import jax.numpy as jnp
from jax import lax
from jax.experimental import pallas as pl
from jax.experimental.pallas import tpu as pltpu
from jax.experimental.pallas import tpu_sc as plsc

N = 10000
E = 320000
NFEAT = 128
NHID = 64
H = 2 * NHID
MAX_LOGSTD = 10.0

NC = 2
NS = 16
NW = NC * NS
CHUNK = 128
EPAD = NW * CHUNK * ((E + NW * CHUNK - 1) // (NW * CHUNK))
TPE = EPAD // NW
NCH = TPE // CHUNK
NACC = 10240
RPT = NACC // NS
BLK = 128
NBLK = NACC // BLK

_mesh = plsc.VectorSubcoreMesh(core_axis_name="c", subcore_axis_name="s")


@functools.partial(
    pl.kernel,
    out_type=jax.ShapeDtypeStruct((NC, NACC, H), jnp.float32),
    mesh=_mesh,
    scratch_types=[
        pltpu.VMEM_SHARED((NACC, H), jnp.float32),
        pltpu.VMEM((CHUNK,), jnp.int32),
        pltpu.VMEM((CHUNK, H), jnp.float32),
    ],
)
def _deg_kernel(dst_hbm, ones_hbm, zeros_hbm, out_hbm, acc, idx_v, ones_v):
    c = lax.axis_index("c")
    s = lax.axis_index("s")
    wid = s * NC + c
    pltpu.sync_copy(zeros_hbm, acc.at[pl.ds(s * RPT, RPT)])
    pltpu.sync_copy(ones_hbm, ones_v)
    plsc.subcore_barrier()

    def body(j, carry):
        base = wid * TPE + j * CHUNK
        pltpu.sync_copy(dst_hbm.at[pl.ds(base, CHUNK)], idx_v)
        pltpu.sync_copy(ones_v, acc.at[idx_v], add=True)
        return carry

    lax.fori_loop(0, NCH, body, 0)
    plsc.subcore_barrier()
    pltpu.sync_copy(acc.at[pl.ds(s * RPT, RPT)],
                    out_hbm.at[c, pl.ds(s * RPT, RPT)])


@functools.partial(
    pl.kernel,
    out_type=jax.ShapeDtypeStruct((NC, NACC, H), jnp.float32),
    mesh=_mesh,
    scratch_types=[
        pltpu.VMEM_SHARED((NACC, H), jnp.float32),
        pltpu.VMEM((CHUNK,), jnp.int32),
        pltpu.VMEM((CHUNK,), jnp.int32),
        pltpu.VMEM((CHUNK, H), jnp.float32),
        pltpu.SemaphoreType.DMA,
    ],
)
def _adj_kernel(y_hbm, src_hbm, dst_hbm, zeros_hbm, out_hbm,
                acc, sidx, didx, rows, gsem):
    c = lax.axis_index("c")
    s = lax.axis_index("s")
    wid = s * NC + c
    pltpu.sync_copy(zeros_hbm, acc.at[pl.ds(s * RPT, RPT)])
    plsc.subcore_barrier()

    def body(j, carry):
        base = wid * TPE + j * CHUNK
        pltpu.sync_copy(src_hbm.at[pl.ds(base, CHUNK)], sidx)
        pltpu.sync_copy(dst_hbm.at[pl.ds(base, CHUNK)], didx)
        pltpu.async_copy(y_hbm.at[sidx], rows, gsem).wait()
        pltpu.sync_copy(rows, acc.at[didx], add=True)
        return carry

    lax.fori_loop(0, NCH, body, 0)
    plsc.subcore_barrier()
    pltpu.sync_copy(acc.at[pl.ds(s * RPT, RPT)],
                    out_hbm.at[c, pl.ds(s * RPT, RPT)])


def _tc1_body(x_ref, w_ref, b_ref, d_ref, y1_ref, dinv_ref):
    i = pl.program_id(0)
    deg = d_ref[0, :, :1] + d_ref[1, :, :1] + 1.0
    rows = i * BLK + lax.broadcasted_iota(jnp.int32, (BLK, 1), 0)
    dinv = jnp.where(rows < N, lax.rsqrt(deg), 0.0)
    h0 = jnp.dot(x_ref[...], w_ref[...], preferred_element_type=jnp.float32)
    y1_ref[...] = dinv * (h0 + b_ref[...])
    dinv_ref[...] = dinv


def _tc2_body(u_ref, y1_ref, dinv_ref, w_ref, b_ref, y2_ref):
    dinv = dinv_ref[...]
    s1 = dinv * (u_ref[0] + u_ref[1] + y1_ref[...])
    h = jnp.dot(s1, w_ref[...], preferred_element_type=jnp.float32)
    h = jnp.maximum(h + b_ref[...], 0.0)
    y2_ref[...] = dinv * h


def _tc3_body(u_ref, y2_ref, dinv_ref, w_ref, b_ref, eps_ref, z_ref):
    s2 = dinv_ref[...] * (u_ref[0] + u_ref[1] + y2_ref[...])
    o = jnp.dot(s2, w_ref[...], preferred_element_type=jnp.float32)
    o = o + b_ref[...]
    mu = o[:, :NHID]
    ls = jnp.minimum(o[:, NHID:], MAX_LOGSTD)
    z_ref[...] = mu + eps_ref[...] * jnp.exp(ls)


def _row_spec(width):
    return pl.BlockSpec((BLK, width), lambda i: (i, 0))


def _full_spec(shape):
    ndim = len(shape)
    return pl.BlockSpec(shape, lambda i: (0,) * ndim)


def _parts_spec(width):
    return pl.BlockSpec((NC, BLK, width), lambda i: (0, i, 0))


def kernel(x, edge_index, lin_W, lin_b, W1, b1, Wmu, bmu, Wls, bls, eps):
    src = edge_index[0]
    dst = edge_index[1]
    pad = jnp.full((EPAD - E,), N, dtype=jnp.int32)
    src_p = jnp.concatenate([src, pad])
    dst_p = jnp.concatenate([dst, pad])
    x_p = jnp.pad(x, ((0, NACC - N), (0, 0)))
    eps_p = jnp.pad(eps, ((0, NACC - N), (0, 0)))
    ones16 = jnp.ones((CHUNK, H), jnp.float32)
    zrows = jnp.zeros((RPT, H), jnp.float32)
    W2 = jnp.concatenate([Wmu, Wls], axis=1)
    b2 = jnp.concatenate([bmu, bls])[None, :]
    b1r = b1[None, :]
    linbr = lin_b[None, :]

    deg_parts = _deg_kernel(dst_p, ones16, zrows)

    y1, dinv = pl.pallas_call(
        _tc1_body,
        grid=(NBLK,),
        in_specs=[_row_spec(NFEAT), _full_spec((NFEAT, H)),
                  _full_spec((1, H)), _parts_spec(H)],
        out_specs=[_row_spec(H), _row_spec(1)],
        out_shape=[jax.ShapeDtypeStruct((NACC, H), jnp.float32),
                   jax.ShapeDtypeStruct((NACC, 1), jnp.float32)],
    )(x_p, lin_W, linbr, deg_parts)

    u1 = _adj_kernel(y1, src_p, dst_p, zrows)

    y2 = pl.pallas_call(
        _tc2_body,
        grid=(NBLK,),
        in_specs=[_parts_spec(H), _row_spec(H), _row_spec(1),
                  _full_spec((H, H)), _full_spec((1, H))],
        out_specs=_row_spec(H),
        out_shape=jax.ShapeDtypeStruct((NACC, H), jnp.float32),
    )(u1, y1, dinv, W1, b1r)

    u2 = _adj_kernel(y2, src_p, dst_p, zrows)

    z = pl.pallas_call(
        _tc3_body,
        grid=(NBLK,),
        in_specs=[_parts_spec(H), _row_spec(H), _row_spec(1),
                  _full_spec((H, H)), _full_spec((1, H)), _row_spec(NHID)],
        out_specs=_row_spec(NHID),
        out_shape=jax.ShapeDtypeStruct((NACC, NHID), jnp.float32),
    )(u2, y2, dinv, W2, b2, eps_p)

    return z[:N]
